# Initial kernel scaffold; baseline (speedup 1.0000x reference)
import jax
import jax.numpy as jnp
from jax.experimental import pallas as pl


def kernel(x, codebook):
    raise NotImplementedError("write your pallas kernel here")


# speedup vs baseline: 3.5645x; 3.5645x over previous
import functools

import jax
import jax.numpy as jnp
from jax.experimental import pallas as pl

NB = 512
CD = 64


def _vq_body(x_ref, cb_ref, out_ref, cnt_ref, loss_ref):
    first = (pl.program_id(0) == 0) & (pl.program_id(1) == 0)
    xt = x_ref[0]
    cb = cb_ref[...]

    xn2 = jnp.sum(xt * xt, axis=0, keepdims=True)
    norm = jnp.sqrt(xn2)
    xf = xt / jnp.maximum(norm, 1e-12)
    xfn2 = jnp.sum(xf * xf, axis=0, keepdims=True)

    dots = jax.lax.dot_general(cb, xf, (((1,), (0,)), ((), ())),
                               preferred_element_type=jnp.float32)
    cn2 = jnp.sum(cb * cb, axis=1, keepdims=True)
    score = cn2 - 2.0 * dots

    idx = jnp.argmin(score, axis=0)
    onehot = (jax.lax.broadcasted_iota(jnp.int32, score.shape, 0)
              == idx[None, :]).astype(jnp.float32)

    xd = jax.lax.dot_general(cb, onehot, (((0,), (0,)), ((), ())),
                             preferred_element_type=jnp.float32)
    out_ref[0] = xd

    cnt = jnp.sum(onehot, axis=1, keepdims=True)
    mind = jnp.min(score, axis=0, keepdims=True) + xfn2
    lsum = jnp.sum(mind).reshape(1, 1)

    @pl.when(first)
    def _():
        cnt_ref[...] = cnt
        loss_ref[...] = lsum

    @pl.when(jnp.logical_not(first))
    def _():
        cnt_ref[...] = cnt_ref[...] + cnt
        loss_ref[...] = loss_ref[...] + lsum


@functools.partial(jax.jit, static_argnames=("tt",))
def _vq(x, codebook, tt=512):
    n, w, t = x.shape
    out, cnt, lsum = pl.pallas_call(
        _vq_body,
        grid=(n, t // tt),
        in_specs=[
            pl.BlockSpec((1, w, tt), lambda i, j: (i, 0, j)),
            pl.BlockSpec((NB, CD), lambda i, j: (0, 0)),
        ],
        out_specs=[
            pl.BlockSpec((1, w, tt), lambda i, j: (i, 0, j)),
            pl.BlockSpec((NB, 1), lambda i, j: (0, 0)),
            pl.BlockSpec((1, 1), lambda i, j: (0, 0)),
        ],
        out_shape=[
            jax.ShapeDtypeStruct((n, w, t), jnp.float32),
            jax.ShapeDtypeStruct((NB, 1), jnp.float32),
            jax.ShapeDtypeStruct((1, 1), jnp.float32),
        ],
    )(x, codebook)
    ntok = n * t
    count = cnt[:, 0]
    prob = count / jnp.sum(count)
    perplexity = jnp.exp(-jnp.sum(prob * jnp.log(prob + 1e-7)))
    commit_loss = lsum[0, 0] / (ntok * w)
    return out, commit_loss, perplexity


def kernel(x, codebook):
    return _vq(x, codebook)
